# initial kernel scaffold (unmeasured)
import jax
import jax.numpy as jnp
from jax import lax
from jax.experimental import pallas as pl
from jax.experimental.pallas import tpu as pltpu

N_DEV = 8
SQ = 2048
DM = 1024
HQ = 8
DH = 128
WIN = 128
QB = 256
KW = 512
NBLK = SQ // QB
SCALE = 0.08838834764831843
NEG = -1e30

CHAIN_NEXT = (-1, 2, 3, 7, 5, 6, -1, -1)


def kernel(x, Wq, K_ext, V_ext, Wo):
    def body(x_ref, wq_ref, k_ref, v_ref, wo_ref, out_ref,
             tailk_ref, tailv_ref, send_sems, recv_sems, tsend, trecv):
        my = lax.axis_index("i")
        is_root = my == 0
        nexts = jnp.array(CHAIN_NEXT, dtype=jnp.int32)
        nxt = nexts[my]
        has_next = nxt >= 0

        tail_rdmas = []
        for src, dst, i in ((k_ref, tailk_ref, 0), (v_ref, tailv_ref, 1)):
            tail_rdmas.append(pltpu.make_async_remote_copy(
                src_ref=src.at[0, pl.ds(0, WIN), :, :],
                dst_ref=dst,
                send_sem=tsend.at[i],
                recv_sem=trecv.at[i],
                device_id=(0,),
                device_id_type=pl.DeviceIdType.MESH,
            ))

        @pl.when(my == 1)
        def _():
            for r in tail_rdmas:
                r.start()

        @pl.when(is_root)
        def _():
            wq = wq_ref[:, :].astype(jnp.bfloat16)
            wo = wo_ref[:, :].astype(jnp.bfloat16)
            sends = []
            for k in range(NBLK):
                w0 = max(0, k * QB - WIN)
                if k == NBLK - 1:
                    for r in tail_rdmas:
                        r.wait_recv()
                    kwin = jnp.concatenate(
                        [k_ref[0, w0:SQ, :, :], tailk_ref[:, :, :]], axis=0)
                    vwin = jnp.concatenate(
                        [v_ref[0, w0:SQ, :, :], tailv_ref[:, :, :]], axis=0)
                else:
                    kwin = k_ref[0, w0:w0 + KW, :, :]
                    vwin = v_ref[0, w0:w0 + KW, :, :]

                xb = x_ref[0, k * QB:(k + 1) * QB, :].astype(jnp.bfloat16)
                qb = lax.dot(xb, wq, preferred_element_type=jnp.float32)

                qi = k * QB + lax.broadcasted_iota(jnp.int32, (QB, KW), 0)
                kj = w0 + lax.broadcasted_iota(jnp.int32, (QB, KW), 1)
                valid = jnp.abs(qi - kj) <= WIN

                ctx_cols = []
                for h in range(HQ):
                    qh = qb[:, h * DH:(h + 1) * DH].astype(jnp.bfloat16)
                    kh = kwin[:, h, :].astype(jnp.bfloat16)
                    s = lax.dot_general(
                        qh, kh, (((1,), (1,)), ((), ())),
                        preferred_element_type=jnp.float32) * SCALE
                    s = jnp.where(valid, s, NEG)
                    m = jnp.max(s, axis=1, keepdims=True)
                    p = jnp.exp(s - m)
                    pw = (p / jnp.sum(p, axis=1, keepdims=True))
                    vh = vwin[:, h, :].astype(jnp.bfloat16)
                    ctx_cols.append(lax.dot(
                        pw.astype(jnp.bfloat16), vh,
                        preferred_element_type=jnp.float32))
                ctx = jnp.concatenate(ctx_cols, axis=1)
                outb = lax.dot(ctx.astype(jnp.bfloat16), wo,
                               preferred_element_type=jnp.float32)
                out_ref[0, k * QB:(k + 1) * QB, :] = outb

                for tgt, col in ((1, 0), (4, 1)):
                    r = pltpu.make_async_remote_copy(
                        src_ref=out_ref.at[0, pl.ds(k * QB, QB), :],
                        dst_ref=out_ref.at[0, pl.ds(k * QB, QB), :],
                        send_sem=send_sems.at[k, col],
                        recv_sem=recv_sems.at[k],
                        device_id=(tgt,),
                        device_id_type=pl.DeviceIdType.MESH,
                    )
                    r.start()
                    sends.append(r)
            for r in sends:
                r.wait_send()

        @pl.when(jnp.logical_not(is_root))
        def _():
            for k in range(NBLK):
                r = pltpu.make_async_remote_copy(
                    src_ref=out_ref.at[0, pl.ds(k * QB, QB), :],
                    dst_ref=out_ref.at[0, pl.ds(k * QB, QB), :],
                    send_sem=send_sems.at[k, 0],
                    recv_sem=recv_sems.at[k],
                    device_id=(nxt,),
                    device_id_type=pl.DeviceIdType.MESH,
                )
                r.wait_recv()

                @pl.when(has_next)
                def _():
                    r.start()
                    r.wait_send()

        @pl.when(my == 1)
        def _():
            for r in tail_rdmas:
                r.wait_send()

    return pl.pallas_call(
        body,
        out_shape=jax.ShapeDtypeStruct((1, SQ, DM), jnp.float32),
        in_specs=[pl.BlockSpec(memory_space=pltpu.VMEM)] * 5,
        out_specs=pl.BlockSpec(memory_space=pltpu.VMEM),
        scratch_shapes=[
            pltpu.VMEM((WIN, HQ, DH), jnp.float32),
            pltpu.VMEM((WIN, HQ, DH), jnp.float32),
            pltpu.SemaphoreType.DMA((NBLK, 2)),
            pltpu.SemaphoreType.DMA((NBLK,)),
            pltpu.SemaphoreType.DMA((2,)),
            pltpu.SemaphoreType.DMA((2,)),
        ],
    )(x, Wq, K_ext, V_ext, Wo)


# baseline (device time: 164710 ns/iter reference)
import jax
import jax.numpy as jnp
from jax import lax
from jax.experimental import pallas as pl
from jax.experimental.pallas import tpu as pltpu

N_DEV = 8
SQ = 2048
DM = 1024
HQ = 8
DH = 128
WIN = 128
QB = 256
KW = 512
NBLK = SQ // QB
SCALE = 0.08838834764831843
NEG = -1e30

CHAIN_NEXT = (-1, 2, 3, 7, 5, 6, -1, -1)


def kernel(x, Wq, K_ext, V_ext, Wo):
    def body(x_ref, wq_ref, k_ref, v_ref, wo_ref, out_ref,
             tailk_ref, tailv_ref, send_sems, recv_sems, tsend, trecv):
        my = lax.axis_index("i")
        is_root = my == 0
        nxt = jnp.where(my == 3, 7, my + 1).astype(jnp.int32)
        has_next = jnp.logical_and(my >= 1, my <= 5)

        tail_rdmas = []
        for src, dst, i in ((k_ref, tailk_ref, 0), (v_ref, tailv_ref, 1)):
            tail_rdmas.append(pltpu.make_async_remote_copy(
                src_ref=src.at[0, pl.ds(0, WIN), :, :],
                dst_ref=dst,
                send_sem=tsend.at[i],
                recv_sem=trecv.at[i],
                device_id=(0,),
                device_id_type=pl.DeviceIdType.MESH,
            ))

        @pl.when(my == 1)
        def _():
            for r in tail_rdmas:
                r.start()

        @pl.when(is_root)
        def _():
            wq = wq_ref[:, :].astype(jnp.bfloat16)
            wo = wo_ref[:, :].astype(jnp.bfloat16)
            sends = []
            for k in range(NBLK):
                w0 = max(0, k * QB - WIN)
                if k == NBLK - 1:
                    for r in tail_rdmas:
                        r.wait_recv()
                    kwin = jnp.concatenate(
                        [k_ref[0, w0:SQ, :, :], tailk_ref[:, :, :]], axis=0)
                    vwin = jnp.concatenate(
                        [v_ref[0, w0:SQ, :, :], tailv_ref[:, :, :]], axis=0)
                else:
                    kwin = k_ref[0, w0:w0 + KW, :, :]
                    vwin = v_ref[0, w0:w0 + KW, :, :]

                xb = x_ref[0, k * QB:(k + 1) * QB, :].astype(jnp.bfloat16)
                qb = lax.dot(xb, wq, preferred_element_type=jnp.float32)

                qi = k * QB + lax.broadcasted_iota(jnp.int32, (QB, KW), 0)
                kj = w0 + lax.broadcasted_iota(jnp.int32, (QB, KW), 1)
                valid = jnp.abs(qi - kj) <= WIN

                ctx_cols = []
                for h in range(HQ):
                    qh = qb[:, h * DH:(h + 1) * DH].astype(jnp.bfloat16)
                    kh = kwin[:, h, :].astype(jnp.bfloat16)
                    s = lax.dot_general(
                        qh, kh, (((1,), (1,)), ((), ())),
                        preferred_element_type=jnp.float32) * SCALE
                    s = jnp.where(valid, s, NEG)
                    m = jnp.max(s, axis=1, keepdims=True)
                    p = jnp.exp(s - m)
                    pw = (p / jnp.sum(p, axis=1, keepdims=True))
                    vh = vwin[:, h, :].astype(jnp.bfloat16)
                    ctx_cols.append(lax.dot(
                        pw.astype(jnp.bfloat16), vh,
                        preferred_element_type=jnp.float32))
                ctx = jnp.concatenate(ctx_cols, axis=1)
                outb = lax.dot(ctx.astype(jnp.bfloat16), wo,
                               preferred_element_type=jnp.float32)
                out_ref[0, k * QB:(k + 1) * QB, :] = outb

                for tgt, col in ((1, 0), (4, 1)):
                    r = pltpu.make_async_remote_copy(
                        src_ref=out_ref.at[0, pl.ds(k * QB, QB), :],
                        dst_ref=out_ref.at[0, pl.ds(k * QB, QB), :],
                        send_sem=send_sems.at[k, col],
                        recv_sem=recv_sems.at[k],
                        device_id=(tgt,),
                        device_id_type=pl.DeviceIdType.MESH,
                    )
                    r.start()
                    sends.append(r)
            for r in sends:
                r.wait_send()

        @pl.when(jnp.logical_not(is_root))
        def _():
            for k in range(NBLK):
                r = pltpu.make_async_remote_copy(
                    src_ref=out_ref.at[0, pl.ds(k * QB, QB), :],
                    dst_ref=out_ref.at[0, pl.ds(k * QB, QB), :],
                    send_sem=send_sems.at[k, 0],
                    recv_sem=recv_sems.at[k],
                    device_id=(nxt,),
                    device_id_type=pl.DeviceIdType.MESH,
                )
                r.wait_recv()

                @pl.when(has_next)
                def _():
                    r.start()
                    r.wait_send()

        @pl.when(my == 1)
        def _():
            for r in tail_rdmas:
                r.wait_send()

    return pl.pallas_call(
        body,
        out_shape=jax.ShapeDtypeStruct((1, SQ, DM), jnp.float32),
        in_specs=[pl.BlockSpec(memory_space=pltpu.VMEM)] * 5,
        out_specs=pl.BlockSpec(memory_space=pltpu.VMEM),
        scratch_shapes=[
            pltpu.VMEM((WIN, HQ, DH), jnp.float32),
            pltpu.VMEM((WIN, HQ, DH), jnp.float32),
            pltpu.SemaphoreType.DMA((NBLK, 2)),
            pltpu.SemaphoreType.DMA((NBLK,)),
            pltpu.SemaphoreType.DMA((2,)),
            pltpu.SemaphoreType.DMA((2,)),
        ],
        compiler_params=pltpu.CompilerParams(
            vmem_limit_bytes=100 * 1024 * 1024,
        ),
    )(x, Wq, K_ext, V_ext, Wo)
